# baseline (speedup 1.0000x reference)
import functools

import jax
import jax.numpy as jnp
from jax import lax
from jax.experimental import pallas as pl
from jax.experimental.pallas import tpu as pltpu
from jax.experimental.pallas import tpu_sc as plsc

_NC = 2
_NS = 16
_L = 16


def _argmax_body(text_ref, query_ref, idx_ref, *, q, t):
    s = lax.dot_general(text_ref[0], query_ref[0], (((1,), (1,)), ((), ())))
    rowmax = jnp.max(s, axis=1, keepdims=True)
    eqb = s >= rowmax
    eq_f = eqb.astype(jnp.float32)
    ii = lax.broadcasted_iota(jnp.int32, (q, q), 0)
    jj = lax.broadcasted_iota(jnp.int32, (q, q), 1)
    tstrict = (ii < jj).astype(jnp.float32)
    excl = lax.dot_general(eq_f, tstrict, (((1,), (0,)), ((), ())))
    first = jnp.where(excl == 0.0, eq_f, 0.0)
    qrow = lax.broadcasted_iota(jnp.int32, (1, q), 1).astype(jnp.float32)
    idxr = lax.dot_general(qrow, first, (((1,), (1,)), ((), ())))
    boff = lax.rem(pl.program_id(0), 8) * 128
    idx_ref[0] = idxr.astype(jnp.int32) + boff


def _tc_cluster_ids(query, text):
    b, q, d = query.shape
    t = text.shape[1]
    idx = pl.pallas_call(
        functools.partial(_argmax_body, q=q, t=t),
        grid=(b,),
        in_specs=[
            pl.BlockSpec((1, t, d), lambda i: (i, 0, 0)),
            pl.BlockSpec((1, q, d), lambda i: (i, 0, 0)),
        ],
        out_specs=pl.BlockSpec((1, 1, t), lambda i: (i, 0, 0)),
        out_shape=jax.ShapeDtypeStruct((b, 1, t), jnp.int32),
    )(text, query)
    return idx.reshape(b * t // 128, 128)


def _sc_segment_sort(idx2d, *, b, q, t):
    nw = _NC * _NS
    rows_per_w = idx2d.shape[0] // nw
    toks_per_w = rows_per_w * 128
    batches_per_core = b // _NC
    nbins = batches_per_core * q
    smooth = jnp.float32(0.001)

    mesh = plsc.VectorSubcoreMesh(core_axis_name="c", subcore_axis_name="s")

    @functools.partial(
        pl.kernel,
        mesh=mesh,
        out_type=jax.ShapeDtypeStruct((b, q), jnp.int32),
        compiler_params=pltpu.CompilerParams(needs_layout_passes=False),
        scratch_types=[
            pltpu.VMEM((rows_per_w, 128), jnp.int32),
            pltpu.VMEM((rows_per_w, 128), jnp.float32),
            pltpu.VMEM((rows_per_w, 128), jnp.float32),
            pltpu.VMEM((nbins,), jnp.float32),
            pltpu.VMEM((q,), jnp.float32),
            pltpu.VMEM((q,), jnp.float32),
            pltpu.VMEM((q,), jnp.float32),
            pltpu.VMEM((q,), jnp.int32),
            pltpu.VMEM_SHARED((nbins,), jnp.float32),
            pltpu.VMEM_SHARED((nbins,), jnp.float32),
        ],
    )
    def _sc(idx_hbm, out_hbm, idx_v, w_v, on_v, z_v, wv, cv, o_v, out_v,
            bins_w, bins_c):
        cid = lax.axis_index("c")
        sid = lax.axis_index("s")
        wid = cid * _NS + sid
        half = lax.rem(wid, 2)

        pltpu.sync_copy(idx_hbm.at[pl.ds(wid * rows_per_w, rows_per_w)],
                        idx_v)

        lanes = lax.iota(jnp.int32, _L)
        for r in range(rows_per_w):
            base = t - (half * rows_per_w + r) * 128
            for k in range(128 // _L):
                vals = (base - k * _L) - lanes
                w_v[r, pl.ds(k * _L, _L)] = vals.astype(jnp.float32)
                on_v[r, pl.ds(k * _L, _L)] = jnp.full((_L,), 1.0,
                                                      jnp.float32)

        @pl.when(sid == 0)
        def _zero():
            for k in range(nbins // _L):
                z_v[pl.ds(k * _L, _L)] = jnp.zeros((_L,), jnp.float32)
            pltpu.sync_copy(z_v, bins_w)
            pltpu.sync_copy(z_v, bins_c)

        plsc.subcore_barrier()

        for r in range(rows_per_w):
            pltpu.sync_copy(w_v.at[r], bins_w.at[idx_v.at[r]], add=True)
            pltpu.sync_copy(on_v.at[r], bins_c.at[idx_v.at[r]], add=True)

        plsc.subcore_barrier()

        @pl.when(half == 0)
        def _rank():
            lb = lax.rem(wid // 2, batches_per_core)
            pltpu.sync_copy(bins_w.at[pl.ds(lb * q, q)], wv)
            pltpu.sync_copy(bins_c.at[pl.ds(lb * q, q)], cv)
            nchunk = q // _L
            for a in range(nchunk):
                sl = pl.ds(a * _L, _L)
                o_v[sl] = q - wv[sl] / (cv[sl] + smooth)
            o_chunks = [o_v[pl.ds(a * _L, _L)] for a in range(nchunk)]
            gidx = [a * _L + lanes for a in range(nchunk)]
            ranks = [jnp.zeros((_L,), jnp.int32) for _ in range(nchunk)]
            for jc in range(nchunk):
                oc = o_chunks[jc]
                for jl in range(_L):
                    j = jc * _L + jl
                    bj = oc[jnp.full((_L,), jl, jnp.int32)]
                    for a in range(nchunk):
                        hit = (bj < o_chunks[a]) | (
                            (bj == o_chunks[a]) & (j < gidx[a]))
                        ranks[a] = ranks[a] + hit.astype(jnp.int32)
            for a in range(nchunk):
                plsc.store_scatter(out_v, [ranks[a]], gidx[a])
            batch = wid // 2
            pltpu.sync_copy(out_v, out_hbm.at[batch])

    return _sc(idx2d)


def kernel(query, text):
    b, q, d = query.shape
    t = text.shape[1]
    idx2d = _tc_cluster_ids(query, text)
    return _sc_segment_sort(idx2d, b=b, q=q, t=t)


# speedup vs baseline: 1.0365x; 1.0365x over previous
import functools

import jax
import jax.numpy as jnp
from jax import lax
from jax.experimental import pallas as pl
from jax.experimental.pallas import tpu as pltpu
from jax.experimental.pallas import tpu_sc as plsc

_NC = 2
_NS = 16
_L = 16


def _argmax_body(text_ref, query_ref, idx_ref, *, q, t):
    s = lax.dot_general(text_ref[0], query_ref[0], (((1,), (1,)), ((), ())))
    rowmax = jnp.max(s, axis=1, keepdims=True)
    eqb = s >= rowmax
    eq_f = eqb.astype(jnp.float32)
    ii = lax.broadcasted_iota(jnp.int32, (q, q), 0)
    jj = lax.broadcasted_iota(jnp.int32, (q, q), 1)
    tstrict = (ii < jj).astype(jnp.float32)
    excl = lax.dot_general(eq_f, tstrict, (((1,), (0,)), ((), ())))
    first = jnp.where(excl == 0.0, eq_f, 0.0)
    qrow = lax.broadcasted_iota(jnp.int32, (1, q), 1).astype(jnp.float32)
    idxr = lax.dot_general(qrow, first, (((1,), (1,)), ((), ())))
    boff = lax.rem(pl.program_id(0), 8) * 128
    idx_ref[0] = idxr.astype(jnp.int32) + boff


def _tc_cluster_ids(query, text):
    b, q, d = query.shape
    t = text.shape[1]
    idx = pl.pallas_call(
        functools.partial(_argmax_body, q=q, t=t),
        grid=(b,),
        in_specs=[
            pl.BlockSpec((1, t, d), lambda i: (i, 0, 0)),
            pl.BlockSpec((1, q, d), lambda i: (i, 0, 0)),
        ],
        out_specs=pl.BlockSpec((1, 1, t), lambda i: (i, 0, 0)),
        out_shape=jax.ShapeDtypeStruct((b, 1, t), jnp.int32),
    )(text, query)
    return idx.reshape(b * t // 128, 128)


def _sc_segment_sort(idx2d, *, b, q, t):
    nw = _NC * _NS
    rows_per_w = idx2d.shape[0] // nw
    toks_per_w = rows_per_w * 128
    batches_per_core = b // _NC
    nbins = batches_per_core * q
    smooth = jnp.float32(0.001)

    mesh = plsc.VectorSubcoreMesh(core_axis_name="c", subcore_axis_name="s")

    @functools.partial(
        pl.kernel,
        mesh=mesh,
        out_type=jax.ShapeDtypeStruct((b, q), jnp.int32),
        compiler_params=pltpu.CompilerParams(needs_layout_passes=False),
        scratch_types=[
            pltpu.VMEM((rows_per_w, 128), jnp.int32),
            pltpu.VMEM((rows_per_w, 128), jnp.float32),
            pltpu.VMEM((rows_per_w, 128), jnp.float32),
            pltpu.VMEM((nbins,), jnp.float32),
            pltpu.VMEM((q,), jnp.float32),
            pltpu.VMEM((q,), jnp.float32),
            pltpu.VMEM((q,), jnp.float32),
            pltpu.VMEM((q,), jnp.int32),
            pltpu.VMEM_SHARED((nbins,), jnp.float32),
            pltpu.VMEM_SHARED((nbins,), jnp.float32),
            pltpu.SemaphoreType.DMA,
        ],
    )
    def _sc(idx_hbm, out_hbm, idx_v, w_v, on_v, z_v, wv, cv, o_v, out_v,
            bins_w, bins_c, dsem):
        cid = lax.axis_index("c")
        sid = lax.axis_index("s")
        wid = cid * _NS + sid
        half = lax.rem(wid, 2)

        pltpu.sync_copy(idx_hbm.at[pl.ds(wid * rows_per_w, rows_per_w)],
                        idx_v)

        lanes = lax.iota(jnp.int32, _L)
        for r in range(rows_per_w):
            base = t - (half * rows_per_w + r) * 128
            for k in range(128 // _L):
                vals = (base - k * _L) - lanes
                w_v[r, pl.ds(k * _L, _L)] = vals.astype(jnp.float32)
                on_v[r, pl.ds(k * _L, _L)] = jnp.full((_L,), 1.0,
                                                      jnp.float32)

        @pl.when(sid == 0)
        def _zero():
            for k in range(nbins // _L):
                z_v[pl.ds(k * _L, _L)] = jnp.zeros((_L,), jnp.float32)
            pltpu.sync_copy(z_v, bins_w)
            pltpu.sync_copy(z_v, bins_c)

        plsc.subcore_barrier()

        handles = []
        for r in range(rows_per_w):
            handles.append(pltpu.async_copy(
                w_v.at[r], bins_w.at[idx_v.at[r]], dsem, add=True))
            handles.append(pltpu.async_copy(
                on_v.at[r], bins_c.at[idx_v.at[r]], dsem, add=True))
        for h in handles:
            h.wait()

        plsc.subcore_barrier()

        @pl.when(half == 0)
        def _rank():
            lb = lax.rem(wid // 2, batches_per_core)
            pltpu.sync_copy(bins_w.at[pl.ds(lb * q, q)], wv)
            pltpu.sync_copy(bins_c.at[pl.ds(lb * q, q)], cv)
            nchunk = q // _L
            for a in range(nchunk):
                sl = pl.ds(a * _L, _L)
                o_v[sl] = q - wv[sl] / (cv[sl] + smooth)
            o_chunks = [o_v[pl.ds(a * _L, _L)] for a in range(nchunk)]
            gidx = [a * _L + lanes for a in range(nchunk)]
            ranks = [jnp.zeros((_L,), jnp.int32) for _ in range(nchunk)]
            for jc in range(nchunk):
                oc = o_chunks[jc]
                for jl in range(_L):
                    j = jc * _L + jl
                    bj = oc[jnp.full((_L,), jl, jnp.int32)]
                    for a in range(nchunk):
                        hit = (bj < o_chunks[a]) | (
                            (bj == o_chunks[a]) & (j < gidx[a]))
                        ranks[a] = ranks[a] + hit.astype(jnp.int32)
            for a in range(nchunk):
                plsc.store_scatter(out_v, [ranks[a]], gidx[a])
            batch = wid // 2
            pltpu.sync_copy(out_v, out_hbm.at[batch])

    return _sc(idx2d)


def kernel(query, text):
    b, q, d = query.shape
    t = text.shape[1]
    idx2d = _tc_cluster_ids(query, text)
    return _sc_segment_sort(idx2d, b=b, q=q, t=t)
